# Initial kernel scaffold; baseline (speedup 1.0000x reference)
import jax
import jax.numpy as jnp
from jax.experimental import pallas as pl


def kernel(x, edge_index, pos, params):
    raise NotImplementedError("write your pallas kernel here")


# speedup vs baseline: 3.3184x; 3.3184x over previous
import jax
import jax.numpy as jnp
import numpy as np
from jax import lax
from jax.experimental import pallas as pl
from jax.experimental.pallas import tpu as pltpu
from jax.experimental.pallas import tpu_sc as plsc

N = 10000
E = 320000
H = 128

NC = 2
NS = 16
NW = NC * NS
EW = E // NW
C = 80
NCH_W = EW // C
GRP = 5
NGRP = NCH_W // GRP
GE = GRP * C

_F32 = jnp.float32
_I32 = jnp.int32


def _mesh():
    return plsc.VectorSubcoreMesh(
        core_axis_name="c", subcore_axis_name="s", num_cores=NC, num_subcores=NS
    )


def _wid():
    return lax.axis_index("s") * NC + lax.axis_index("c")


def _sc_gather_one(table, idx3d):
    def body(tab, idx, out, idxv, buf, sem):
        wid = _wid()
        pltpu.sync_copy(idx.at[wid], idxv)
        base = wid * EW

        def grp_body(gi, carry):
            ofs = base + gi * GE
            hs = [
                pltpu.async_copy(
                    tab.at[idxv.at[gi * GRP + j]], buf.at[pl.ds(j * C, C)], sem
                )
                for j in range(GRP)
            ]
            for hdl in hs:
                hdl.wait()
            pltpu.sync_copy(buf, out.at[pl.ds(ofs, GE)])
            return carry

        lax.fori_loop(0, NGRP, grp_body, 0)

    fn = pl.kernel(
        body,
        out_type=jax.ShapeDtypeStruct((E, H), _F32),
        mesh=_mesh(),
        compiler_params=pltpu.CompilerParams(needs_layout_passes=False),
        scratch_types=[
            pltpu.VMEM((NCH_W, C), _I32),
            pltpu.VMEM((GE, H), _F32),
            pltpu.SemaphoreType.DMA,
        ],
    )
    return fn(table, idx3d)


def _sc_rel(pos_t, row3d, col3d):
    def body(pt, ridx, cidx, out, tab, idxr, idxc, ob):
        wid = _wid()
        pltpu.sync_copy(pt, tab)
        pltpu.sync_copy(ridx.at[wid], idxr)
        pltpu.sync_copy(cidx.at[wid], idxc)

        zeros16 = jnp.zeros((16,), _F32)
        for j in range(C // 16):
            r = lax.iota(_I32, 16) + j * 16
            plsc.store_scatter(ob, [r, jnp.full((16,), 3, _I32)], zeros16)

        def chunk(i, carry):
            ofs = wid * EW + i * C
            ii = jnp.full((16,), i, _I32)
            for j in range(C // 16):
                r = lax.iota(_I32, 16) + j * 16
                ir = plsc.load_gather(idxr, [ii, r])
                ic = plsc.load_gather(idxc, [ii, r])
                for k in range(3):
                    kk = jnp.full((16,), k, _I32)
                    pk = plsc.load_gather(tab, [kk, ir])
                    qk = plsc.load_gather(tab, [kk, ic])
                    plsc.store_scatter(ob, [r, kk], qk - pk)
            pltpu.sync_copy(ob, out.at[pl.ds(ofs, C)])
            return carry

        lax.fori_loop(0, NCH_W, chunk, 0)

    fn = pl.kernel(
        body,
        out_type=jax.ShapeDtypeStruct((E, 4), _F32),
        mesh=_mesh(),
        compiler_params=pltpu.CompilerParams(needs_layout_passes=False),
        scratch_types=[
            pltpu.VMEM((4, N), _F32),
            pltpu.VMEM((NCH_W, C), _I32),
            pltpu.VMEM((NCH_W, C), _I32),
            pltpu.VMEM((C, 4), _F32),
        ],
    )
    return fn(pos_t, row3d, col3d)


def _sc_scatter(coef, frame, col3d, zeros4n):
    def body(coef_h, frame_h, cidx, z4, out, idxb, cb, fb, acc):
        wid = _wid()
        pltpu.sync_copy(z4, acc)
        pltpu.sync_copy(cidx.at[wid], idxb)

        k0 = jnp.full((16,), 0, _I32)
        k1 = jnp.full((16,), 1, _I32)
        k2 = jnp.full((16,), 2, _I32)

        def chunk(i, carry):
            ofs = wid * EW + i * C
            pltpu.sync_copy(coef_h.at[pl.ds(ofs, C)], cb)
            pltpu.sync_copy(frame_h.at[pl.ds(ofs, C)], fb)
            ii = jnp.full((16,), i, _I32)
            for j in range(C // 16):
                r = lax.iota(_I32, 16) + j * 16
                ids = plsc.load_gather(idxb, [ii, r])

                def ldf(k):
                    return plsc.load_gather(fb, [r, jnp.full((16,), k, _I32)])

                fs = plsc.load_gather(cb, [r, k0])
                fv0 = plsc.load_gather(cb, [r, k1])
                fv1 = plsc.load_gather(cb, [r, k2])
                e1x, e1y, e1z = ldf(4), ldf(5), ldf(6)
                e2x, e2y = ldf(8), ldf(9)
                e3x, e3y, e3z = ldf(12), ldf(13), ldf(14)
                fx = fs * e1x + fv0 * e2x + fv1 * e3x
                fy = fs * e1y + fv0 * e2y + fv1 * e3y
                fz = fs * e1z + fv1 * e3z
                plsc.addupdate_scatter(acc, [k0, ids], fx)
                plsc.addupdate_scatter(acc, [k1, ids], fy)
                plsc.addupdate_scatter(acc, [k2, ids], fz)
            return carry

        lax.fori_loop(0, NCH_W, chunk, 0)
        pltpu.sync_copy(acc, out.at[wid])

    fn = pl.kernel(
        body,
        out_type=jax.ShapeDtypeStruct((NW, 4, N), _F32),
        mesh=_mesh(),
        compiler_params=pltpu.CompilerParams(needs_layout_passes=False),
        scratch_types=[
            pltpu.VMEM((NCH_W, C), _I32),
            pltpu.VMEM((C, 4), _F32),
            pltpu.VMEM((C, 16), _F32),
            pltpu.VMEM((4, N), _F32),
        ],
    )
    return fn(coef, frame, col3d, zeros4n)


def _mlp2_body(x_ref, w1_ref, b1_ref, w2_ref, b2_ref, o_ref):
    t = jnp.dot(x_ref[...], w1_ref[...], preferred_element_type=_F32) + b1_ref[...]
    t = jnp.maximum(t, 0.0)
    o_ref[...] = jnp.dot(t, w2_ref[...], preferred_element_type=_F32) + b2_ref[...]


def _tc_mlp2(x, w1, b1, w2, b2, bn):
    n, fi = x.shape
    fh = w1.shape[1]
    fo = w2.shape[1]
    return pl.pallas_call(
        _mlp2_body,
        grid=(n // bn,),
        in_specs=[
            pl.BlockSpec((bn, fi), lambda i: (i, 0)),
            pl.BlockSpec((fi, fh), lambda i: (0, 0)),
            pl.BlockSpec((1, fh), lambda i: (0, 0)),
            pl.BlockSpec((fh, fo), lambda i: (0, 0)),
            pl.BlockSpec((1, fo), lambda i: (0, 0)),
        ],
        out_specs=pl.BlockSpec((bn, fo), lambda i: (i, 0)),
        out_shape=jax.ShapeDtypeStruct((n, fo), _F32),
    )(x, w1, b1.reshape(1, -1), w2, b2.reshape(1, -1))


def _frame_body(rel_ref, ones_ref, s12_ref, perm_ref, r1_ref, r2_ref, o_ref):
    f32 = _F32
    rel = rel_ref[...]
    ones44 = ones_ref[...]
    s12 = s12_ref[...]
    permj = perm_ref[...]
    r1j = r1_ref[...]
    r2j = r2_ref[...]

    d2 = jnp.dot(rel * rel, ones44, preferred_element_type=f32)
    e1 = rel / (jnp.sqrt(d2) + 1e-6)
    n2 = jnp.dot(e1 * e1, s12, preferred_element_type=f32)
    e2 = jnp.dot(e1, permj, preferred_element_type=f32) / (jnp.sqrt(n2) + 1e-6)
    e3 = jnp.dot(e1, r1j, preferred_element_type=f32) * jnp.dot(
        e2, r2j, preferred_element_type=f32
    ) - jnp.dot(e1, r2j, preferred_element_type=f32) * jnp.dot(
        e2, r1j, preferred_element_type=f32
    )
    o_ref[...] = jnp.concatenate([rel, e1, e2, e3], axis=1)


def _tc_frame(rel, te):
    ones44 = jnp.ones((4, 4), _F32)
    s12 = jnp.asarray(
        np.array([[1, 1, 1, 1], [1, 1, 1, 1], [0, 0, 0, 0], [0, 0, 0, 0]], np.float32)
    )
    perm = np.zeros((4, 4), np.float32)
    perm[1, 0] = 1.0
    perm[0, 1] = -1.0
    r1 = np.zeros((4, 4), np.float32)
    r1[1, 0] = 1.0
    r1[2, 1] = 1.0
    r1[0, 2] = 1.0
    r2 = np.zeros((4, 4), np.float32)
    r2[2, 0] = 1.0
    r2[0, 1] = 1.0
    r2[1, 2] = 1.0
    full = lambda: pl.BlockSpec((4, 4), lambda i: (0, 0))
    return pl.pallas_call(
        _frame_body,
        grid=(E // te,),
        in_specs=[
            pl.BlockSpec((te, 4), lambda i: (i, 0)),
            full(), full(), full(), full(), full(),
        ],
        out_specs=pl.BlockSpec((te, 16), lambda i: (i, 0)),
        out_shape=jax.ShapeDtypeStruct((E, 16), _F32),
    )(rel, ones44, s12, jnp.asarray(perm), jnp.asarray(r1), jnp.asarray(r2))


def _edge_body(
    gr_ref, gc_ref, rel_ref, wa_ref, wb_ref, wc_ref, b1_ref,
    w2s_ref, b2s_ref, w2v_ref, b2v_ref, w3a_ref, w3b_ref, b3_ref, o_ref
):
    t = (
        jnp.dot(gr_ref[...], wa_ref[...], preferred_element_type=_F32)
        + jnp.dot(gc_ref[...], wb_ref[...], preferred_element_type=_F32)
        + jnp.dot(rel_ref[:, 0:4], wc_ref[...], preferred_element_type=_F32)
        + b1_ref[...]
    )
    t = jnp.maximum(t, 0.0)
    s = jnp.maximum(
        jnp.dot(t[:, :H], w2s_ref[...], preferred_element_type=_F32) + b2s_ref[...], 0.0
    )
    v = jnp.maximum(
        jnp.dot(t[:, H:], w2v_ref[...], preferred_element_type=_F32) + b2v_ref[...], 0.0
    )
    o_ref[...] = (
        jnp.dot(s, w3a_ref[...], preferred_element_type=_F32)
        + jnp.dot(v, w3b_ref[...], preferred_element_type=_F32)
        + b3_ref[...]
    )


def _tc_edge(gr, gc, frame, wa, wb, wc4, b1, w2s, b2s, w2v, b2v, w3a, w3b, b3, te):
    full = lambda shape: pl.BlockSpec(shape, lambda i: (0, 0))
    return pl.pallas_call(
        _edge_body,
        grid=(E // te,),
        in_specs=[
            pl.BlockSpec((te, H), lambda i: (i, 0)),
            pl.BlockSpec((te, H), lambda i: (i, 0)),
            pl.BlockSpec((te, 16), lambda i: (i, 0)),
            full((H, 2 * H)),
            full((H, 2 * H)),
            full((4, 2 * H)),
            full((1, 2 * H)),
            full((H, H)),
            full((1, H)),
            full((H, H)),
            full((1, H)),
            full((H, 4)),
            full((H, 4)),
            full((1, 4)),
        ],
        out_specs=pl.BlockSpec((te, 4), lambda i: (i, 0)),
        out_shape=jax.ShapeDtypeStruct((E, 4), _F32),
    )(
        gr, gc, frame, wa, wb, wc4, b1.reshape(1, -1),
        w2s, b2s.reshape(1, -1), w2v, b2v.reshape(1, -1),
        w3a, w3b, b3.reshape(1, -1),
    )


def _reduce_body(ag_ref, o_ref):
    a = jnp.sum(ag_ref[...], axis=0)
    o_ref[...] = a.T


def _tc_reduce_t(aggr):
    return pl.pallas_call(
        _reduce_body,
        grid=(1,),
        in_specs=[pl.BlockSpec((NW, 4, N), lambda i: (0, 0, 0))],
        out_specs=pl.BlockSpec((N, 4), lambda i: (0, 0)),
        out_shape=jax.ShapeDtypeStruct((N, 4), _F32),
    )(aggr)


def _update_body(h_ref, ag_ref, wh_ref, wa_ref, b1_ref, w2_ref, b2_ref, o_ref):
    t = (
        jnp.dot(h_ref[...], wh_ref[...], preferred_element_type=_F32)
        + jnp.dot(ag_ref[...], wa_ref[...], preferred_element_type=_F32)
        + b1_ref[...]
    )
    t = jnp.maximum(t, 0.0)
    o_ref[...] = h_ref[...] + jnp.dot(t, w2_ref[...], preferred_element_type=_F32) + b2_ref[...]


def _tc_update(h, aggrt, wh, wa4, b1, w2, b2, bn):
    full = lambda shape: pl.BlockSpec(shape, lambda *_: (0,) * len(shape))
    return pl.pallas_call(
        _update_body,
        grid=(N // bn,),
        in_specs=[
            pl.BlockSpec((bn, H), lambda i: (i, 0)),
            pl.BlockSpec((bn, 4), lambda i: (i, 0)),
            full((H, H)),
            full((4, H)),
            full((1, H)),
            full((H, H)),
            full((1, H)),
        ],
        out_specs=pl.BlockSpec((bn, H), lambda i: (i, 0)),
        out_shape=jax.ShapeDtypeStruct((N, H), _F32),
    )(h, aggrt, wh, wa4, b1.reshape(1, -1), w2, b2.reshape(1, -1))


_BN = 2000
_TE = 2560


def kernel(x, edge_index, pos, params):
    bn = _BN
    te = _TE

    x8 = jnp.pad(x, ((0, 0), (0, 2)))
    pos_t = jnp.pad(pos.T, ((0, 1), (0, 0)))
    row3d = edge_index[0].reshape(NW, NCH_W, C)
    col3d = edge_index[1].reshape(NW, NCH_W, C)
    zeros4n = jnp.zeros((4, N), _F32)

    (w_e1, b_e1), (w_e2, b_e2) = params["enc"]
    w_e1p = jnp.pad(w_e1, ((0, 2), (0, 0)))
    h = _tc_mlp2(x8, w_e1p, b_e1, w_e2, b_e2, bn)

    rel = _sc_rel(pos_t, row3d, col3d)
    frame = _tc_frame(rel, te)

    for lp in params["layers"]:
        (ws1, bs1), (ws2, bs2), (ws3, bs3) = lp["scalar"]
        (wv1, bv1), (wv2, bv2), (wv3, bv3) = lp["vector"]
        (wu1, bu1), (wu2, bu2) = lp["update"]

        wa = jnp.concatenate([ws1[:H], wv1[:H]], axis=1)
        wb = jnp.concatenate([ws1[H : 2 * H], wv1[H : 2 * H]], axis=1)
        wc4 = jnp.pad(
            jnp.concatenate([ws1[2 * H :], wv1[2 * H :]], axis=1), ((0, 1), (0, 0))
        )
        b1 = jnp.concatenate([bs1, bv1])
        w3a = jnp.pad(ws3, ((0, 0), (0, 3)))
        w3b = jnp.pad(wv3, ((0, 0), (1, 1)))
        b3 = jnp.concatenate([bs3, bv3, jnp.zeros((1,), _F32)])
        wu1h = wu1[:H]
        wu1a = jnp.pad(wu1[H:], ((0, 1), (0, 0)))

        gr = _sc_gather_one(h, row3d)
        gc = _sc_gather_one(h, col3d)
        coef = _tc_edge(
            gr, gc, frame, wa, wb, wc4, b1, ws2, bs2, wv2, bv2, w3a, w3b, b3, te
        )
        aggr = _sc_scatter(coef, frame, col3d, zeros4n)
        h = _tc_update(h, _tc_reduce_t(aggr), wu1h, wu1a, bu1, wu2, bu2, bn)

    (w_d1, b_d1), (w_d2, b_d2) = params["dec"]
    return _tc_mlp2(h, w_d1, b_d1, w_d2, b_d2, bn)


# speedup vs baseline: 3.3744x; 1.0169x over previous
import jax
import jax.numpy as jnp
import numpy as np
from jax import lax
from jax.experimental import pallas as pl
from jax.experimental.pallas import tpu as pltpu
from jax.experimental.pallas import tpu_sc as plsc

N = 10000
E = 320000
H = 128

NC = 2
NS = 16
NW = NC * NS
EW = E // NW
C = 80
NCH_W = EW // C
GRP = 5
NGRP = NCH_W // GRP
GE = GRP * C

_F32 = jnp.float32
_I32 = jnp.int32


def _mesh():
    return plsc.VectorSubcoreMesh(
        core_axis_name="c", subcore_axis_name="s", num_cores=NC, num_subcores=NS
    )


def _wid():
    return lax.axis_index("s") * NC + lax.axis_index("c")


def _sc_gather_one(table, idx3d):
    def body(tab, idx, out, idxv, buf_a, buf_b, gsem_a, gsem_b, osem_a, osem_b):
        wid = _wid()
        pltpu.sync_copy(idx.at[wid], idxv)
        base = wid * EW

        def fire(gi, buf, sem):
            for j in range(GRP):
                pltpu.async_copy(
                    tab.at[idxv.at[gi * GRP + j]], buf.at[pl.ds(j * C, C)], sem
                )

        def drain(buf, sem):
            pltpu.make_async_copy(tab.at[pl.ds(0, GE)], buf, sem).wait()

        fire(0, buf_a, gsem_a)

        def outer(go, carry):
            gi0 = 2 * go
            gi1 = 2 * go + 1

            @pl.when(go > 0)
            def _():
                drain(buf_b, osem_b)

            @pl.when(gi1 < NGRP)
            def _():
                fire(gi1, buf_b, gsem_b)

            drain(buf_a, gsem_a)
            pltpu.async_copy(buf_a, out.at[pl.ds(base + gi0 * GE, GE)], osem_a)

            @pl.when(gi0 + 2 < NGRP)
            def _():
                drain(buf_a, osem_a)
                fire(gi0 + 2, buf_a, gsem_a)

            @pl.when(gi1 < NGRP)
            def _():
                drain(buf_b, gsem_b)
                pltpu.async_copy(buf_b, out.at[pl.ds(base + gi1 * GE, GE)], osem_b)

            return carry

        lax.fori_loop(0, (NGRP + 1) // 2, outer, 0)
        drain(buf_a, osem_a)

    fn = pl.kernel(
        body,
        out_type=jax.ShapeDtypeStruct((E, H), _F32),
        mesh=_mesh(),
        compiler_params=pltpu.CompilerParams(needs_layout_passes=False),
        scratch_types=[
            pltpu.VMEM((NCH_W, C), _I32),
            pltpu.VMEM((GE, H), _F32),
            pltpu.VMEM((GE, H), _F32),
            pltpu.SemaphoreType.DMA,
            pltpu.SemaphoreType.DMA,
            pltpu.SemaphoreType.DMA,
            pltpu.SemaphoreType.DMA,
        ],
    )
    return fn(table, idx3d)


def _sc_rel(pos_t, row3d, col3d):
    def body(pt, ridx, cidx, out, tab, idxr, idxc, ob):
        wid = _wid()
        pltpu.sync_copy(pt, tab)
        pltpu.sync_copy(ridx.at[wid], idxr)
        pltpu.sync_copy(cidx.at[wid], idxc)

        zeros16 = jnp.zeros((16,), _F32)
        for j in range(C // 16):
            r = lax.iota(_I32, 16) + j * 16
            plsc.store_scatter(ob, [r, jnp.full((16,), 3, _I32)], zeros16)

        def chunk(i, carry):
            ofs = wid * EW + i * C
            ii = jnp.full((16,), i, _I32)
            for j in range(C // 16):
                r = lax.iota(_I32, 16) + j * 16
                ir = plsc.load_gather(idxr, [ii, r])
                ic = plsc.load_gather(idxc, [ii, r])
                for k in range(3):
                    kk = jnp.full((16,), k, _I32)
                    pk = plsc.load_gather(tab, [kk, ir])
                    qk = plsc.load_gather(tab, [kk, ic])
                    plsc.store_scatter(ob, [r, kk], qk - pk)
            pltpu.sync_copy(ob, out.at[pl.ds(ofs, C)])
            return carry

        lax.fori_loop(0, NCH_W, chunk, 0)

    fn = pl.kernel(
        body,
        out_type=jax.ShapeDtypeStruct((E, 4), _F32),
        mesh=_mesh(),
        compiler_params=pltpu.CompilerParams(needs_layout_passes=False),
        scratch_types=[
            pltpu.VMEM((4, N), _F32),
            pltpu.VMEM((NCH_W, C), _I32),
            pltpu.VMEM((NCH_W, C), _I32),
            pltpu.VMEM((C, 4), _F32),
        ],
    )
    return fn(pos_t, row3d, col3d)


def _sc_scatter(coef, frame, col3d, zeros4n):
    def body(coef_h, frame_h, cidx, z4, out, idxb, cb, fb, acc):
        wid = _wid()
        pltpu.sync_copy(z4, acc)
        pltpu.sync_copy(cidx.at[wid], idxb)

        k0 = jnp.full((16,), 0, _I32)
        k1 = jnp.full((16,), 1, _I32)
        k2 = jnp.full((16,), 2, _I32)

        def chunk(i, carry):
            ofs = wid * EW + i * C
            pltpu.sync_copy(coef_h.at[pl.ds(ofs, C)], cb)
            pltpu.sync_copy(frame_h.at[pl.ds(ofs, C)], fb)
            ii = jnp.full((16,), i, _I32)
            for j in range(C // 16):
                r = lax.iota(_I32, 16) + j * 16
                ids = plsc.load_gather(idxb, [ii, r])

                def ldf(k):
                    return plsc.load_gather(fb, [r, jnp.full((16,), k, _I32)])

                fs = plsc.load_gather(cb, [r, k0])
                fv0 = plsc.load_gather(cb, [r, k1])
                fv1 = plsc.load_gather(cb, [r, k2])
                e1x, e1y, e1z = ldf(4), ldf(5), ldf(6)
                e2x, e2y = ldf(8), ldf(9)
                e3x, e3y, e3z = ldf(12), ldf(13), ldf(14)
                fx = fs * e1x + fv0 * e2x + fv1 * e3x
                fy = fs * e1y + fv0 * e2y + fv1 * e3y
                fz = fs * e1z + fv1 * e3z
                plsc.addupdate_scatter(acc, [k0, ids], fx)
                plsc.addupdate_scatter(acc, [k1, ids], fy)
                plsc.addupdate_scatter(acc, [k2, ids], fz)
            return carry

        lax.fori_loop(0, NCH_W, chunk, 0)
        pltpu.sync_copy(acc, out.at[wid])

    fn = pl.kernel(
        body,
        out_type=jax.ShapeDtypeStruct((NW, 4, N), _F32),
        mesh=_mesh(),
        compiler_params=pltpu.CompilerParams(needs_layout_passes=False),
        scratch_types=[
            pltpu.VMEM((NCH_W, C), _I32),
            pltpu.VMEM((C, 4), _F32),
            pltpu.VMEM((C, 16), _F32),
            pltpu.VMEM((4, N), _F32),
        ],
    )
    return fn(coef, frame, col3d, zeros4n)


def _mlp2_body(x_ref, w1_ref, b1_ref, w2_ref, b2_ref, o_ref):
    t = jnp.dot(x_ref[...], w1_ref[...], preferred_element_type=_F32) + b1_ref[...]
    t = jnp.maximum(t, 0.0)
    o_ref[...] = jnp.dot(t, w2_ref[...], preferred_element_type=_F32) + b2_ref[...]


def _tc_mlp2(x, w1, b1, w2, b2, bn):
    n, fi = x.shape
    fh = w1.shape[1]
    fo = w2.shape[1]
    return pl.pallas_call(
        _mlp2_body,
        grid=(n // bn,),
        in_specs=[
            pl.BlockSpec((bn, fi), lambda i: (i, 0)),
            pl.BlockSpec((fi, fh), lambda i: (0, 0)),
            pl.BlockSpec((1, fh), lambda i: (0, 0)),
            pl.BlockSpec((fh, fo), lambda i: (0, 0)),
            pl.BlockSpec((1, fo), lambda i: (0, 0)),
        ],
        out_specs=pl.BlockSpec((bn, fo), lambda i: (i, 0)),
        out_shape=jax.ShapeDtypeStruct((n, fo), _F32),
    )(x, w1, b1.reshape(1, -1), w2, b2.reshape(1, -1))


def _frame_body(rel_ref, ones_ref, s12_ref, perm_ref, r1_ref, r2_ref, o_ref):
    f32 = _F32
    rel = rel_ref[...]
    ones44 = ones_ref[...]
    s12 = s12_ref[...]
    permj = perm_ref[...]
    r1j = r1_ref[...]
    r2j = r2_ref[...]

    d2 = jnp.dot(rel * rel, ones44, preferred_element_type=f32)
    e1 = rel / (jnp.sqrt(d2) + 1e-6)
    n2 = jnp.dot(e1 * e1, s12, preferred_element_type=f32)
    e2 = jnp.dot(e1, permj, preferred_element_type=f32) / (jnp.sqrt(n2) + 1e-6)
    e3 = jnp.dot(e1, r1j, preferred_element_type=f32) * jnp.dot(
        e2, r2j, preferred_element_type=f32
    ) - jnp.dot(e1, r2j, preferred_element_type=f32) * jnp.dot(
        e2, r1j, preferred_element_type=f32
    )
    o_ref[...] = jnp.concatenate([rel, e1, e2, e3], axis=1)


def _tc_frame(rel, te):
    ones44 = jnp.ones((4, 4), _F32)
    s12 = jnp.asarray(
        np.array([[1, 1, 1, 1], [1, 1, 1, 1], [0, 0, 0, 0], [0, 0, 0, 0]], np.float32)
    )
    perm = np.zeros((4, 4), np.float32)
    perm[1, 0] = 1.0
    perm[0, 1] = -1.0
    r1 = np.zeros((4, 4), np.float32)
    r1[1, 0] = 1.0
    r1[2, 1] = 1.0
    r1[0, 2] = 1.0
    r2 = np.zeros((4, 4), np.float32)
    r2[2, 0] = 1.0
    r2[0, 1] = 1.0
    r2[1, 2] = 1.0
    full = lambda: pl.BlockSpec((4, 4), lambda i: (0, 0))
    return pl.pallas_call(
        _frame_body,
        grid=(E // te,),
        in_specs=[
            pl.BlockSpec((te, 4), lambda i: (i, 0)),
            full(), full(), full(), full(), full(),
        ],
        out_specs=pl.BlockSpec((te, 16), lambda i: (i, 0)),
        out_shape=jax.ShapeDtypeStruct((E, 16), _F32),
    )(rel, ones44, s12, jnp.asarray(perm), jnp.asarray(r1), jnp.asarray(r2))


def _edge_body(
    gr_ref, gc_ref, rel_ref, wa_ref, wb_ref, wc_ref, b1_ref,
    w2s_ref, b2s_ref, w2v_ref, b2v_ref, w3a_ref, w3b_ref, b3_ref, o_ref
):
    t = (
        jnp.dot(gr_ref[...], wa_ref[...], preferred_element_type=_F32)
        + jnp.dot(gc_ref[...], wb_ref[...], preferred_element_type=_F32)
        + jnp.dot(rel_ref[:, 0:4], wc_ref[...], preferred_element_type=_F32)
        + b1_ref[...]
    )
    t = jnp.maximum(t, 0.0)
    s = jnp.maximum(
        jnp.dot(t[:, :H], w2s_ref[...], preferred_element_type=_F32) + b2s_ref[...], 0.0
    )
    v = jnp.maximum(
        jnp.dot(t[:, H:], w2v_ref[...], preferred_element_type=_F32) + b2v_ref[...], 0.0
    )
    o_ref[...] = (
        jnp.dot(s, w3a_ref[...], preferred_element_type=_F32)
        + jnp.dot(v, w3b_ref[...], preferred_element_type=_F32)
        + b3_ref[...]
    )


def _tc_edge(gr, gc, frame, wa, wb, wc4, b1, w2s, b2s, w2v, b2v, w3a, w3b, b3, te):
    full = lambda shape: pl.BlockSpec(shape, lambda i: (0, 0))
    return pl.pallas_call(
        _edge_body,
        grid=(E // te,),
        in_specs=[
            pl.BlockSpec((te, H), lambda i: (i, 0)),
            pl.BlockSpec((te, H), lambda i: (i, 0)),
            pl.BlockSpec((te, 16), lambda i: (i, 0)),
            full((H, 2 * H)),
            full((H, 2 * H)),
            full((4, 2 * H)),
            full((1, 2 * H)),
            full((H, H)),
            full((1, H)),
            full((H, H)),
            full((1, H)),
            full((H, 4)),
            full((H, 4)),
            full((1, 4)),
        ],
        out_specs=pl.BlockSpec((te, 4), lambda i: (i, 0)),
        out_shape=jax.ShapeDtypeStruct((E, 4), _F32),
    )(
        gr, gc, frame, wa, wb, wc4, b1.reshape(1, -1),
        w2s, b2s.reshape(1, -1), w2v, b2v.reshape(1, -1),
        w3a, w3b, b3.reshape(1, -1),
    )


def _reduce_body(ag_ref, o_ref):
    a = jnp.sum(ag_ref[...], axis=0)
    o_ref[...] = a.T


def _tc_reduce_t(aggr):
    return pl.pallas_call(
        _reduce_body,
        grid=(1,),
        in_specs=[pl.BlockSpec((NW, 4, N), lambda i: (0, 0, 0))],
        out_specs=pl.BlockSpec((N, 4), lambda i: (0, 0)),
        out_shape=jax.ShapeDtypeStruct((N, 4), _F32),
    )(aggr)


def _update_body(h_ref, ag_ref, wh_ref, wa_ref, b1_ref, w2_ref, b2_ref, o_ref):
    t = (
        jnp.dot(h_ref[...], wh_ref[...], preferred_element_type=_F32)
        + jnp.dot(ag_ref[...], wa_ref[...], preferred_element_type=_F32)
        + b1_ref[...]
    )
    t = jnp.maximum(t, 0.0)
    o_ref[...] = h_ref[...] + jnp.dot(t, w2_ref[...], preferred_element_type=_F32) + b2_ref[...]


def _tc_update(h, aggrt, wh, wa4, b1, w2, b2, bn):
    full = lambda shape: pl.BlockSpec(shape, lambda *_: (0,) * len(shape))
    return pl.pallas_call(
        _update_body,
        grid=(N // bn,),
        in_specs=[
            pl.BlockSpec((bn, H), lambda i: (i, 0)),
            pl.BlockSpec((bn, 4), lambda i: (i, 0)),
            full((H, H)),
            full((4, H)),
            full((1, H)),
            full((H, H)),
            full((1, H)),
        ],
        out_specs=pl.BlockSpec((bn, H), lambda i: (i, 0)),
        out_shape=jax.ShapeDtypeStruct((N, H), _F32),
    )(h, aggrt, wh, wa4, b1.reshape(1, -1), w2, b2.reshape(1, -1))


_BN = 2000
_TE = 2560


def kernel(x, edge_index, pos, params):
    bn = _BN
    te = _TE

    x8 = jnp.pad(x, ((0, 0), (0, 2)))
    pos_t = jnp.pad(pos.T, ((0, 1), (0, 0)))
    row3d = edge_index[0].reshape(NW, NCH_W, C)
    col3d = edge_index[1].reshape(NW, NCH_W, C)
    zeros4n = jnp.zeros((4, N), _F32)

    (w_e1, b_e1), (w_e2, b_e2) = params["enc"]
    w_e1p = jnp.pad(w_e1, ((0, 2), (0, 0)))
    h = _tc_mlp2(x8, w_e1p, b_e1, w_e2, b_e2, bn)

    rel = _sc_rel(pos_t, row3d, col3d)
    frame = _tc_frame(rel, te)

    for lp in params["layers"]:
        (ws1, bs1), (ws2, bs2), (ws3, bs3) = lp["scalar"]
        (wv1, bv1), (wv2, bv2), (wv3, bv3) = lp["vector"]
        (wu1, bu1), (wu2, bu2) = lp["update"]

        wa = jnp.concatenate([ws1[:H], wv1[:H]], axis=1)
        wb = jnp.concatenate([ws1[H : 2 * H], wv1[H : 2 * H]], axis=1)
        wc4 = jnp.pad(
            jnp.concatenate([ws1[2 * H :], wv1[2 * H :]], axis=1), ((0, 1), (0, 0))
        )
        b1 = jnp.concatenate([bs1, bv1])
        w3a = jnp.pad(ws3, ((0, 0), (0, 3)))
        w3b = jnp.pad(wv3, ((0, 0), (1, 1)))
        b3 = jnp.concatenate([bs3, bv3, jnp.zeros((1,), _F32)])
        wu1h = wu1[:H]
        wu1a = jnp.pad(wu1[H:], ((0, 1), (0, 0)))

        gr = _sc_gather_one(h, row3d)
        gc = _sc_gather_one(h, col3d)
        coef = _tc_edge(
            gr, gc, frame, wa, wb, wc4, b1, ws2, bs2, wv2, bv2, w3a, w3b, b3, te
        )
        aggr = _sc_scatter(coef, frame, col3d, zeros4n)
        h = _tc_update(h, _tc_reduce_t(aggr), wu1h, wu1a, bu1, wu2, bu2, bn)

    (w_d1, b_d1), (w_d2, b_d2) = params["dec"]
    return _tc_mlp2(h, w_d1, b_d1, w_d2, b_d2, bn)
